# baseline (speedup 1.0000x reference)
import functools
import math

import jax
import jax.numpy as jnp
from jax import lax
from jax.experimental import pallas as pl
from jax.experimental.pallas import tpu as pltpu
from jax.experimental.pallas import tpu_sc as plsc

NC = 2
NS = 16
K = 128
L = 16


def _k1_body(x_ref, w1_ref, att_ref, x1_ref, a2_ref):
    x1 = jnp.dot(x_ref[...], w1_ref[...], preferred_element_type=jnp.float32)
    x1_ref[...] = x1
    a2_ref[...] = jnp.dot(x1, att_ref[...], preferred_element_type=jnp.float32)


def _k2_body(u0_ref, u1_ref, s_ref, w2_ref, o_ref):
    s = s_ref[...] + 1e-16
    out = jnp.concatenate([u0_ref[...], u1_ref[...]], axis=1) / s
    h1 = jnp.where(out > 0, out, jnp.exp(out) - 1.0)
    o_ref[...] = jnp.dot(h1, w2_ref[...], preferred_element_type=jnp.float32)


def _make_sc_kernel(n_pad, nch, d):
    dh = d // 2
    dw = dh // 2
    mesh = plsc.VectorSubcoreMesh(
        core_axis_name="c", subcore_axis_name="s", num_cores=NC, num_subcores=NS
    )
    rows_per_tile = n_pad // NS
    n_zero_copies = rows_per_tile // K
    srow = n_pad // d
    assert srow <= K and srow % L == 0 and rows_per_tile % K == 0

    @functools.partial(
        pl.kernel,
        mesh=mesh,
        compiler_params=pltpu.CompilerParams(
            needs_layout_passes=False, use_tc_tiling_on_sc=False),
        out_type=(
            jax.ShapeDtypeStruct((NC, n_pad, dh), jnp.float32),
            jax.ShapeDtypeStruct((NC, srow, d), jnp.float32),
        ),
        scratch_types=[
            pltpu.VMEM((4, 2, K), jnp.int32),
            pltpu.VMEM((srow, d), jnp.float32),
            pltpu.VMEM((srow, d), jnp.float32),
            pltpu.VMEM((srow, d), jnp.float32),
            pltpu.VMEM((K, L), jnp.float32),
            pltpu.VMEM((2, K, dw), jnp.int32),
            pltpu.VMEM((2, K, dh), jnp.float32),
            pltpu.VMEM((1, srow), jnp.int32),
            pltpu.VMEM_SHARED((n_pad, dw), jnp.int32),
            pltpu.VMEM_SHARED((n_pad, dh), jnp.float32),
            pltpu.VMEM_SHARED((srow, d), jnp.float32),
            pltpu.SemaphoreType.DMA,
            pltpu.SemaphoreType.DMA,
            pltpu.SemaphoreType.DMA,
            pltpu.SemaphoreType.DMA,
            pltpu.SemaphoreType.DMA,
            pltpu.SemaphoreType.DMA,
        ],
    )
    def sc_edge_kernel(x1p_hbm, asrc_hbm, adst_hbm, eir_hbm,
                       u_out, s_out,
                       idx_ring, asrc_v, adst_v, s_loc, pbc, rows_bf, rows_f,
                       rowidx, xt_sh, u_sh, s_sh,
                       gsem0, gsem1, ssem0, ssem1, isem0, isem1):
        gsems = (gsem0, gsem1)
        ssems = (ssem0, ssem1)
        isems = (isem0, isem1)
        cid = lax.axis_index("c")
        sid = lax.axis_index("s")
        zero16 = jnp.zeros((L,), jnp.float32)
        lane16 = lax.iota(jnp.int32, L)
        himask = jnp.full((L,), -65536, jnp.int32)

        def _zrows(i, _):
            for col in range(dh // L):
                rows_f[0, i, pl.ds(col * L, L)] = zero16
            return 0
        lax.fori_loop(0, K, _zrows, 0)

        def _zs(i, _):
            for col in range(d // L):
                s_loc[i, pl.ds(col * L, L)] = zero16
            return 0
        lax.fori_loop(0, srow, _zs, 0)

        for g in range(srow // L):
            rowidx[0, pl.ds(g * L, L)] = lane16 + g * L

        for c in range(n_zero_copies):
            pltpu.sync_copy(rows_f.at[0],
                            u_sh.at[pl.ds(sid * rows_per_tile + c * K, K)])

        @pl.when(sid == 0)
        def _():
            pltpu.sync_copy(s_loc, s_sh)

        pltpu.sync_copy(asrc_hbm, asrc_v)
        pltpu.sync_copy(adst_hbm, adst_v)
        pltpu.sync_copy(x1p_hbm.at[cid, pl.ds(sid * rows_per_tile, rows_per_tile)],
                        xt_sh.at[pl.ds(sid * rows_per_tile, rows_per_tile)])

        plsc.subcore_barrier()

        def _wait_idx(p):
            pltpu.make_async_copy(
                eir_hbm.at[sid, 0], idx_ring.at[0], isems[p]).wait()

        def _wait_gather(p):
            pltpu.make_async_copy(
                xt_sh.at[idx_ring.at[0, 0]], rows_bf.at[p], gsems[p]).wait()

        def _wait_scatter(p):
            pltpu.make_async_copy(
                rows_f.at[p], u_sh.at[idx_ring.at[0, 1]], ssems[p]).wait()

        pltpu.async_copy(eir_hbm.at[sid, 0], idx_ring.at[0], isems[0])
        pltpu.async_copy(eir_hbm.at[sid, 1], idx_ring.at[1], isems[1])
        _wait_idx(0)
        pltpu.async_copy(xt_sh.at[idx_ring.at[0, 0]], rows_bf.at[0], gsems[0])

        def ring_body(jo, _):
            for b in range(4):
                j = jo * 4 + b
                bn = (b + 1) % 4
                b2 = (b + 2) % 4
                p = b % 2
                pn = (b + 1) % 2

                @pl.when(j >= 2)
                def _():
                    _wait_scatter(p)

                @pl.when(j + 2 < nch)
                def _():
                    pltpu.async_copy(eir_hbm.at[sid, j + 2], idx_ring.at[b2],
                                     isems[p])

                @pl.when(j + 1 < nch)
                def _():
                    _wait_idx(pn)
                    pltpu.async_copy(
                        xt_sh.at[idx_ring.at[bn, 0]], rows_bf.at[pn],
                        gsems[pn])

                _wait_gather(p)

                for g in range(K // L):
                    si = idx_ring[b, 0, pl.ds(g * L, L)]
                    di = idx_ring[b, 1, pl.ds(g * L, L)]
                    av = plsc.load_gather(asrc_v, [si >> 7, si & (d - 1)])
                    bv = plsc.load_gather(adst_v, [di >> 7, di & (d - 1)])
                    sig = 1.0 / (1.0 + jnp.exp(-(av + bv)))
                    pv = jnp.exp(sig)
                    plsc.addupdate_scatter(s_loc, [di >> 7, di & (d - 1)], pv)
                    for m in range(L):
                        plsc.store_scatter(
                            pbc, [lane16 + g * L, jnp.full((L,), m, jnp.int32)],
                            pv)

                def scale_body(k8, _):
                    for r in range(8):
                        k = k8 * 8 + r
                        pk = pbc[k, pl.ds(0, L)]
                        for q in range(dw // L):
                            w = rows_bf[p, k, pl.ds(q * L, L)]
                            lo = plsc.bitcast(w << 16, jnp.float32)
                            hi = plsc.bitcast(w & himask, jnp.float32)
                            rows_f[p, k, pl.ds(q * 2 * L, L)] = lo * pk
                            rows_f[p, k, pl.ds(q * 2 * L + L, L)] = hi * pk
                    return 0
                lax.fori_loop(0, K // 8, scale_body, 0)

                pltpu.async_copy(rows_f.at[p], u_sh.at[idx_ring.at[b, 1]],
                                 ssems[p], add=True)
            return 0

        lax.fori_loop(0, nch // 4, ring_body, 0)

        _wait_scatter(0)
        _wait_scatter(1)

        pltpu.sync_copy(s_loc, s_sh.at[rowidx.at[0]], add=True)
        plsc.subcore_barrier()

        for c in range(n_zero_copies):
            base = sid * rows_per_tile + c * K
            pltpu.sync_copy(u_sh.at[pl.ds(base, K)], u_out.at[cid, pl.ds(base, K)])

        @pl.when(sid == 0)
        def _():
            pltpu.sync_copy(s_sh, s_out.at[cid])

    return sc_edge_kernel


def kernel(x, edge_index, W1, att_src1, att_dst1, W2):
    n, d = x.shape
    e = edge_index.shape[1]
    n_pad = 10240
    bm = 1000
    grid = n // bm

    att_cat = jnp.zeros((d, d), jnp.float32)
    att_cat = att_cat.at[:, 0].set(att_src1.reshape(-1))
    att_cat = att_cat.at[:, 1].set(att_dst1.reshape(-1))

    x1, a2 = pl.pallas_call(
        _k1_body,
        grid=(grid,),
        in_specs=[
            pl.BlockSpec((bm, d), lambda i: (i, 0)),
            pl.BlockSpec((d, d), lambda i: (0, 0)),
            pl.BlockSpec((d, d), lambda i: (0, 0)),
        ],
        out_specs=[
            pl.BlockSpec((bm, d), lambda i: (i, 0)),
            pl.BlockSpec((bm, d), lambda i: (i, 0)),
        ],
        out_shape=[
            jax.ShapeDtypeStruct((n, d), jnp.float32),
            jax.ShapeDtypeStruct((n, d), jnp.float32),
        ],
    )(x, W1, att_cat)

    asrc = jnp.zeros((n_pad,), jnp.float32).at[:n].set(a2[:, 0]).reshape(n_pad // d, d)
    adst = jnp.zeros((n_pad,), jnp.float32).at[:n].set(a2[:, 1]).reshape(n_pad // d, d)

    x1b = x1.astype(jnp.bfloat16)
    x1b = x1b.reshape(n, d // 32, 2, 16).transpose(0, 1, 3, 2).reshape(n, d)
    x1b = jnp.zeros((n_pad, d), jnp.bfloat16).at[:n].set(x1b)
    x1p = jax.lax.bitcast_convert_type(
        x1b.reshape(n_pad, d // 2, 2), jnp.int32)
    x1p = jnp.stack([x1p[:, : d // 4], x1p[:, d // 4:]])

    nch = math.ceil(e / (NS * K))
    nch = ((nch + 3) // 4) * 4
    e_pad = NS * nch * K
    src_r = jnp.concatenate(
        [edge_index[0], jnp.zeros((e_pad - e,), jnp.int32)]
    ).reshape(NS, nch, 1, K)
    dst_r = jnp.concatenate(
        [edge_index[1], jnp.full((e_pad - e,), n, jnp.int32)]
    ).reshape(NS, nch, 1, K)
    ei_r = jnp.concatenate([src_r, dst_r], axis=2)

    u, s = _make_sc_kernel(n_pad, nch, d)(x1p, asrc, adst, ei_r)

    u0 = u[0, :n]
    u1 = u[1, :n]
    s0 = s[0].reshape(n_pad)[:n].reshape(n, 1)

    h2 = pl.pallas_call(
        _k2_body,
        grid=(grid,),
        in_specs=[
            pl.BlockSpec((bm, d // 2), lambda i: (i, 0)),
            pl.BlockSpec((bm, d // 2), lambda i: (i, 0)),
            pl.BlockSpec((bm, 1), lambda i: (i, 0)),
            pl.BlockSpec((d, d), lambda i: (0, 0)),
        ],
        out_specs=pl.BlockSpec((bm, d), lambda i: (i, 0)),
        out_shape=jax.ShapeDtypeStruct((n, d), jnp.float32),
    )(u0, u1, s0, W2)
    return h2


# speedup vs baseline: 1.0925x; 1.0925x over previous
import functools
import math

import jax
import jax.numpy as jnp
from jax import lax
from jax.experimental import pallas as pl
from jax.experimental.pallas import tpu as pltpu
from jax.experimental.pallas import tpu_sc as plsc

NC = 2
NS = 16
K = 128
L = 16


def _k1_body(x_ref, w1_ref, att_ref, xp_ref, a2_ref):
    x1 = jnp.dot(x_ref[...], w1_ref[...], preferred_element_type=jnp.float32)
    a2_ref[...] = jnp.dot(x1, att_ref[...], preferred_element_type=jnp.float32)
    vi = jax.lax.bitcast_convert_type(x1, jnp.int32)
    r = vi + 0x7FFF + ((vi >> 16) & 1)
    words = []
    for q in range(x1.shape[1] // 32):
        lo = jax.lax.shift_right_logical(r[:, q * 32:q * 32 + 16], 16)
        hi = r[:, q * 32 + 16:q * 32 + 32] & jnp.int32(-65536)
        words.append(lo | hi)
    xp_ref[...] = jnp.concatenate(words, axis=1)


def _k2_body(u0_ref, u1_ref, s_ref, w2_ref, o_ref):
    s = s_ref[...] + 1e-16
    out = jnp.concatenate([u0_ref[...], u1_ref[...]], axis=1) / s
    h1 = jnp.where(out > 0, out, jnp.exp(out) - 1.0)
    o_ref[...] = jnp.dot(h1, w2_ref[...], preferred_element_type=jnp.float32)


def _make_sc_kernel(n, n_pad, nch, d):
    dh = d // 2
    dw = dh // 2
    assert n % NS == 0
    mesh = plsc.VectorSubcoreMesh(
        core_axis_name="c", subcore_axis_name="s", num_cores=NC, num_subcores=NS
    )
    rows_per_tile = n_pad // NS
    n_zero_copies = rows_per_tile // K
    srow = n_pad // d
    assert srow <= K and srow % L == 0 and rows_per_tile % K == 0

    @functools.partial(
        pl.kernel,
        mesh=mesh,
        compiler_params=pltpu.CompilerParams(
            needs_layout_passes=False, use_tc_tiling_on_sc=False),
        out_type=(
            jax.ShapeDtypeStruct((NC, n_pad, dh), jnp.float32),
            jax.ShapeDtypeStruct((NC, srow, d), jnp.float32),
        ),
        scratch_types=[
            pltpu.VMEM((4, 2, K), jnp.int32),
            pltpu.VMEM((srow, d), jnp.float32),
            pltpu.VMEM((srow, d), jnp.float32),
            pltpu.VMEM((srow, d), jnp.float32),
            pltpu.VMEM((K, L), jnp.float32),
            pltpu.VMEM((2, K, dw), jnp.int32),
            pltpu.VMEM((2, K, dh), jnp.float32),
            pltpu.VMEM((1, srow), jnp.int32),
            pltpu.VMEM_SHARED((n_pad, dw), jnp.int32),
            pltpu.VMEM_SHARED((n_pad, dh), jnp.float32),
            pltpu.VMEM_SHARED((srow, d), jnp.float32),
            pltpu.SemaphoreType.DMA,
            pltpu.SemaphoreType.DMA,
            pltpu.SemaphoreType.DMA,
            pltpu.SemaphoreType.DMA,
            pltpu.SemaphoreType.DMA,
            pltpu.SemaphoreType.DMA,
        ],
    )
    def sc_edge_kernel(x1p_hbm, asrc_hbm, adst_hbm, eir_hbm,
                       u_out, s_out,
                       idx_ring, asrc_v, adst_v, s_loc, pbc, rows_bf, rows_f,
                       rowidx, xt_sh, u_sh, s_sh,
                       gsem0, gsem1, ssem0, ssem1, isem0, isem1):
        gsems = (gsem0, gsem1)
        ssems = (ssem0, ssem1)
        isems = (isem0, isem1)
        cid = lax.axis_index("c")
        sid = lax.axis_index("s")
        zero16 = jnp.zeros((L,), jnp.float32)
        lane16 = lax.iota(jnp.int32, L)
        himask = jnp.full((L,), -65536, jnp.int32)

        def _zrows(i, _):
            for col in range(dh // L):
                rows_f[0, i, pl.ds(col * L, L)] = zero16
            return 0
        lax.fori_loop(0, K, _zrows, 0)

        def _zs(i, _):
            for col in range(d // L):
                s_loc[i, pl.ds(col * L, L)] = zero16
            return 0
        lax.fori_loop(0, srow, _zs, 0)

        for g in range(srow // L):
            rowidx[0, pl.ds(g * L, L)] = lane16 + g * L

        for c in range(n_zero_copies):
            pltpu.sync_copy(rows_f.at[0],
                            u_sh.at[pl.ds(sid * rows_per_tile + c * K, K)])

        @pl.when(sid == 0)
        def _():
            pltpu.sync_copy(s_loc, s_sh)

        pltpu.sync_copy(asrc_hbm, asrc_v)
        pltpu.sync_copy(adst_hbm, adst_v)
        rst = n // NS
        pltpu.sync_copy(
            x1p_hbm.at[pl.ds(sid * rst, rst), pl.ds(cid * dw, dw)],
            xt_sh.at[pl.ds(sid * rst, rst)])

        plsc.subcore_barrier()

        def _wait_idx(p):
            pltpu.make_async_copy(
                eir_hbm.at[sid, 0], idx_ring.at[0], isems[p]).wait()

        def _wait_gather(p):
            pltpu.make_async_copy(
                xt_sh.at[idx_ring.at[0, 0]], rows_bf.at[p], gsems[p]).wait()

        def _wait_scatter(p):
            pltpu.make_async_copy(
                rows_f.at[p], u_sh.at[idx_ring.at[0, 1]], ssems[p]).wait()

        pltpu.async_copy(eir_hbm.at[sid, 0], idx_ring.at[0], isems[0])
        pltpu.async_copy(eir_hbm.at[sid, 1], idx_ring.at[1], isems[1])
        _wait_idx(0)
        pltpu.async_copy(xt_sh.at[idx_ring.at[0, 0]], rows_bf.at[0], gsems[0])

        def ring_body(jo, _):
            for b in range(4):
                j = jo * 4 + b
                bn = (b + 1) % 4
                b2 = (b + 2) % 4
                p = b % 2
                pn = (b + 1) % 2

                @pl.when(j >= 2)
                def _():
                    _wait_scatter(p)

                @pl.when(j + 2 < nch)
                def _():
                    pltpu.async_copy(eir_hbm.at[sid, j + 2], idx_ring.at[b2],
                                     isems[p])

                @pl.when(j + 1 < nch)
                def _():
                    _wait_idx(pn)
                    pltpu.async_copy(
                        xt_sh.at[idx_ring.at[bn, 0]], rows_bf.at[pn],
                        gsems[pn])

                _wait_gather(p)

                for g in range(K // L):
                    si = idx_ring[b, 0, pl.ds(g * L, L)]
                    di = idx_ring[b, 1, pl.ds(g * L, L)]
                    av = plsc.load_gather(asrc_v, [si >> 7, si & (d - 1)])
                    bv = plsc.load_gather(adst_v, [di >> 7, di & (d - 1)])
                    sig = 1.0 / (1.0 + jnp.exp(-(av + bv)))
                    pv = jnp.exp(sig)
                    plsc.addupdate_scatter(s_loc, [di >> 7, di & (d - 1)], pv)
                    for m in range(L):
                        plsc.store_scatter(
                            pbc, [lane16 + g * L, jnp.full((L,), m, jnp.int32)],
                            pv)

                def scale_body(k8, _):
                    for r in range(8):
                        k = k8 * 8 + r
                        pk = pbc[k, pl.ds(0, L)]
                        for q in range(dw // L):
                            w = rows_bf[p, k, pl.ds(q * L, L)]
                            lo = plsc.bitcast(w << 16, jnp.float32)
                            hi = plsc.bitcast(w & himask, jnp.float32)
                            rows_f[p, k, pl.ds(q * 2 * L, L)] = lo * pk
                            rows_f[p, k, pl.ds(q * 2 * L + L, L)] = hi * pk
                    return 0
                lax.fori_loop(0, K // 8, scale_body, 0)

                pltpu.async_copy(rows_f.at[p], u_sh.at[idx_ring.at[b, 1]],
                                 ssems[p], add=True)
            return 0

        lax.fori_loop(0, nch // 4, ring_body, 0)

        _wait_scatter(0)
        _wait_scatter(1)

        pltpu.sync_copy(s_loc, s_sh.at[rowidx.at[0]], add=True)
        plsc.subcore_barrier()

        for c in range(n_zero_copies):
            base = sid * rows_per_tile + c * K
            pltpu.sync_copy(u_sh.at[pl.ds(base, K)], u_out.at[cid, pl.ds(base, K)])

        @pl.when(sid == 0)
        def _():
            pltpu.sync_copy(s_sh, s_out.at[cid])

    return sc_edge_kernel


def kernel(x, edge_index, W1, att_src1, att_dst1, W2):
    n, d = x.shape
    e = edge_index.shape[1]
    n_pad = 10240
    bm = 1000
    grid = n // bm

    att_cat = jnp.zeros((d, d), jnp.float32)
    att_cat = att_cat.at[:, 0].set(att_src1.reshape(-1))
    att_cat = att_cat.at[:, 1].set(att_dst1.reshape(-1))

    x1p, a2 = pl.pallas_call(
        _k1_body,
        grid=(grid,),
        in_specs=[
            pl.BlockSpec((bm, d), lambda i: (i, 0)),
            pl.BlockSpec((d, d), lambda i: (0, 0)),
            pl.BlockSpec((d, d), lambda i: (0, 0)),
        ],
        out_specs=[
            pl.BlockSpec((bm, d // 2), lambda i: (i, 0)),
            pl.BlockSpec((bm, d), lambda i: (i, 0)),
        ],
        out_shape=[
            jax.ShapeDtypeStruct((n, d // 2), jnp.int32),
            jax.ShapeDtypeStruct((n, d), jnp.float32),
        ],
    )(x, W1, att_cat)

    asrc = jnp.zeros((n_pad,), jnp.float32).at[:n].set(a2[:, 0]).reshape(n_pad // d, d)
    adst = jnp.zeros((n_pad,), jnp.float32).at[:n].set(a2[:, 1]).reshape(n_pad // d, d)

    nch = math.ceil(e / (NS * K))
    nch = ((nch + 3) // 4) * 4
    e_pad = NS * nch * K
    src_r = jnp.concatenate(
        [edge_index[0], jnp.zeros((e_pad - e,), jnp.int32)]
    ).reshape(NS, nch, 1, K)
    dst_r = jnp.concatenate(
        [edge_index[1], jnp.full((e_pad - e,), n, jnp.int32)]
    ).reshape(NS, nch, 1, K)
    ei_r = jnp.concatenate([src_r, dst_r], axis=2)

    u, s = _make_sc_kernel(n, n_pad, nch, d)(x1p, asrc, adst, ei_r)

    u0 = u[0, :n]
    u1 = u[1, :n]
    s0 = s[0].reshape(n_pad)[:n].reshape(n, 1)

    h2 = pl.pallas_call(
        _k2_body,
        grid=(grid,),
        in_specs=[
            pl.BlockSpec((bm, d // 2), lambda i: (i, 0)),
            pl.BlockSpec((bm, d // 2), lambda i: (i, 0)),
            pl.BlockSpec((bm, 1), lambda i: (i, 0)),
            pl.BlockSpec((d, d), lambda i: (0, 0)),
        ],
        out_specs=pl.BlockSpec((bm, d), lambda i: (i, 0)),
        out_shape=jax.ShapeDtypeStruct((n, d), jnp.float32),
    )(u0, u1, s0, W2)
    return h2


# speedup vs baseline: 1.1173x; 1.0228x over previous
import functools
import math

import jax
import jax.numpy as jnp
from jax import lax
from jax.experimental import pallas as pl
from jax.experimental.pallas import tpu as pltpu
from jax.experimental.pallas import tpu_sc as plsc

NC = 2
NS = 16
K = 128
L = 16


def _k1_body(x_ref, w1_ref, att_ref, xp_ref, a2_ref):
    x1 = jnp.dot(x_ref[...], w1_ref[...], preferred_element_type=jnp.float32)
    a2_ref[...] = jnp.dot(x1, att_ref[...], preferred_element_type=jnp.float32)
    vi = jax.lax.bitcast_convert_type(x1, jnp.int32)
    r = vi + 0x7FFF + ((vi >> 16) & 1)
    words = []
    for q in range(x1.shape[1] // 32):
        lo = jax.lax.shift_right_logical(r[:, q * 32:q * 32 + 16], 16)
        hi = r[:, q * 32 + 16:q * 32 + 32] & jnp.int32(-65536)
        words.append(lo | hi)
    xp_ref[...] = jnp.concatenate(words, axis=1)


def _k2_body(u0_ref, u1_ref, s_ref, w2_ref, o_ref):
    s = s_ref[...] + 1e-16
    out = jnp.concatenate([u0_ref[...], u1_ref[...]], axis=1) / s
    h1 = jnp.where(out > 0, out, jnp.exp(out) - 1.0)
    o_ref[...] = jnp.dot(h1, w2_ref[...], preferred_element_type=jnp.float32)


def _make_sc_kernel(n, n_pad, nch, d):
    dh = d // 2
    dw = dh // 2
    assert n % NS == 0
    mesh = plsc.VectorSubcoreMesh(
        core_axis_name="c", subcore_axis_name="s", num_cores=NC, num_subcores=NS
    )
    rows_per_tile = n_pad // NS
    n_zero_copies = rows_per_tile // K
    srow = n_pad // d
    assert srow <= K and srow % L == 0 and rows_per_tile % K == 0

    @functools.partial(
        pl.kernel,
        mesh=mesh,
        compiler_params=pltpu.CompilerParams(
            needs_layout_passes=False, use_tc_tiling_on_sc=False),
        out_type=(
            jax.ShapeDtypeStruct((NC, n_pad, dh), jnp.float32),
            jax.ShapeDtypeStruct((NC, srow, d), jnp.float32),
        ),
        scratch_types=[
            pltpu.VMEM((4, 2, K), jnp.int32),
            pltpu.VMEM((srow, d), jnp.float32),
            pltpu.VMEM((srow, d), jnp.float32),
            pltpu.VMEM((srow, d), jnp.float32),
            pltpu.VMEM((K, L), jnp.float32),
            pltpu.VMEM((2, K, dw), jnp.int32),
            pltpu.VMEM((2, K, dh), jnp.float32),
            pltpu.VMEM((1, srow), jnp.int32),
            pltpu.VMEM_SHARED((n_pad, dw), jnp.int32),
            pltpu.VMEM_SHARED((n_pad, dh), jnp.float32),
            pltpu.VMEM_SHARED((srow, d), jnp.float32),
            pltpu.SemaphoreType.DMA,
            pltpu.SemaphoreType.DMA,
            pltpu.SemaphoreType.DMA,
            pltpu.SemaphoreType.DMA,
            pltpu.SemaphoreType.DMA,
            pltpu.SemaphoreType.DMA,
        ],
    )
    def sc_edge_kernel(x1p_hbm, asrc_hbm, adst_hbm, eir_hbm,
                       u_out, s_out,
                       idx_ring, asrc_v, adst_v, s_loc, pbc, rows_bf, rows_f,
                       rowidx, xt_sh, u_sh, s_sh,
                       gsem0, gsem1, ssem0, ssem1, isem0, isem1):
        gsems = (gsem0, gsem1)
        ssems = (ssem0, ssem1)
        isems = (isem0, isem1)
        cid = lax.axis_index("c")
        sid = lax.axis_index("s")
        zero16 = jnp.zeros((L,), jnp.float32)
        lane16 = lax.iota(jnp.int32, L)
        himask = jnp.full((L,), -65536, jnp.int32)

        def _zrows(i, _):
            for col in range(dh // L):
                rows_f[0, i, pl.ds(col * L, L)] = zero16
            return 0
        lax.fori_loop(0, K, _zrows, 0)

        def _zs(i, _):
            for col in range(d // L):
                s_loc[i, pl.ds(col * L, L)] = zero16
            return 0
        lax.fori_loop(0, srow, _zs, 0)

        for g in range(srow // L):
            rowidx[0, pl.ds(g * L, L)] = lane16 + g * L

        for c in range(n_zero_copies):
            pltpu.sync_copy(rows_f.at[0],
                            u_sh.at[pl.ds(sid * rows_per_tile + c * K, K)])

        @pl.when(sid == 0)
        def _():
            pltpu.sync_copy(s_loc, s_sh)

        pltpu.sync_copy(asrc_hbm, asrc_v)
        pltpu.sync_copy(adst_hbm, adst_v)
        rst = n // NS
        pltpu.sync_copy(
            x1p_hbm.at[pl.ds(sid * rst, rst), pl.ds(cid * dw, dw)],
            xt_sh.at[pl.ds(sid * rst, rst)])

        plsc.subcore_barrier()

        def _wait_idx(p):
            pltpu.make_async_copy(
                eir_hbm.at[sid, 0], idx_ring.at[0], isems[p]).wait()

        def _wait_gather(p):
            pltpu.make_async_copy(
                xt_sh.at[idx_ring.at[0, 0]], rows_bf.at[p], gsems[p]).wait()

        def _wait_scatter(p):
            pltpu.make_async_copy(
                rows_f.at[p], u_sh.at[idx_ring.at[0, 1]], ssems[p]).wait()

        pltpu.async_copy(eir_hbm.at[sid, 0], idx_ring.at[0], isems[0])
        pltpu.async_copy(eir_hbm.at[sid, 1], idx_ring.at[1], isems[1])
        _wait_idx(0)
        pltpu.async_copy(xt_sh.at[idx_ring.at[0, 0]], rows_bf.at[0], gsems[0])

        def ring_body(jo, _):
            for b in range(4):
                j = jo * 4 + b
                bn = (b + 1) % 4
                b2 = (b + 2) % 4
                p = b % 2
                pn = (b + 1) % 2

                @pl.when(j >= 2)
                def _():
                    _wait_scatter(p)

                @pl.when(j + 2 < nch)
                def _():
                    pltpu.async_copy(eir_hbm.at[sid, j + 2], idx_ring.at[b2],
                                     isems[p])

                @pl.when(j + 1 < nch)
                def _():
                    _wait_idx(pn)
                    pltpu.async_copy(
                        xt_sh.at[idx_ring.at[bn, 0]], rows_bf.at[pn],
                        gsems[pn])

                _wait_gather(p)

                for g in range(K // L):
                    si = idx_ring[b, 0, pl.ds(g * L, L)]
                    di = idx_ring[b, 1, pl.ds(g * L, L)]
                    av = plsc.load_gather(asrc_v, [si >> 7, si & (d - 1)])
                    bv = plsc.load_gather(adst_v, [di >> 7, di & (d - 1)])
                    sig = 1.0 / (1.0 + jnp.exp(-(av + bv)))
                    pv = jnp.exp(sig)
                    plsc.addupdate_scatter(s_loc, [di >> 7, di & (d - 1)], pv)
                    for m in range(L):
                        plsc.store_scatter(
                            pbc, [lane16 + g * L, jnp.full((L,), m, jnp.int32)],
                            pv)

                def scale_body(k8, _):
                    for r in range(2):
                        k = k8 * 2 + r
                        pk = pbc[k, pl.ds(0, L)]
                        for q in range(dw // L):
                            w = rows_bf[p, k, pl.ds(q * L, L)]
                            lo = plsc.bitcast(w << 16, jnp.float32)
                            hi = plsc.bitcast(w & himask, jnp.float32)
                            rows_f[p, k, pl.ds(q * 2 * L, L)] = lo * pk
                            rows_f[p, k, pl.ds(q * 2 * L + L, L)] = hi * pk
                    return 0
                lax.fori_loop(0, K // 2, scale_body, 0)

                pltpu.async_copy(rows_f.at[p], u_sh.at[idx_ring.at[b, 1]],
                                 ssems[p], add=True)
            return 0

        lax.fori_loop(0, nch // 4, ring_body, 0)

        _wait_scatter(0)
        _wait_scatter(1)

        pltpu.sync_copy(s_loc, s_sh.at[rowidx.at[0]], add=True)
        plsc.subcore_barrier()

        for c in range(n_zero_copies):
            base = sid * rows_per_tile + c * K
            pltpu.sync_copy(u_sh.at[pl.ds(base, K)], u_out.at[cid, pl.ds(base, K)])

        @pl.when(sid == 0)
        def _():
            pltpu.sync_copy(s_sh, s_out.at[cid])

    return sc_edge_kernel


def kernel(x, edge_index, W1, att_src1, att_dst1, W2):
    n, d = x.shape
    e = edge_index.shape[1]
    n_pad = 10240
    bm = 1000
    grid = n // bm

    att_cat = jnp.zeros((d, d), jnp.float32)
    att_cat = att_cat.at[:, 0].set(att_src1.reshape(-1))
    att_cat = att_cat.at[:, 1].set(att_dst1.reshape(-1))

    x1p, a2 = pl.pallas_call(
        _k1_body,
        grid=(grid,),
        in_specs=[
            pl.BlockSpec((bm, d), lambda i: (i, 0)),
            pl.BlockSpec((d, d), lambda i: (0, 0)),
            pl.BlockSpec((d, d), lambda i: (0, 0)),
        ],
        out_specs=[
            pl.BlockSpec((bm, d // 2), lambda i: (i, 0)),
            pl.BlockSpec((bm, d), lambda i: (i, 0)),
        ],
        out_shape=[
            jax.ShapeDtypeStruct((n, d // 2), jnp.int32),
            jax.ShapeDtypeStruct((n, d), jnp.float32),
        ],
    )(x, W1, att_cat)

    asrc = jnp.zeros((n_pad,), jnp.float32).at[:n].set(a2[:, 0]).reshape(n_pad // d, d)
    adst = jnp.zeros((n_pad,), jnp.float32).at[:n].set(a2[:, 1]).reshape(n_pad // d, d)

    nch = math.ceil(e / (NS * K))
    nch = ((nch + 3) // 4) * 4
    e_pad = NS * nch * K
    src_r = jnp.concatenate(
        [edge_index[0], jnp.zeros((e_pad - e,), jnp.int32)]
    ).reshape(NS, nch, 1, K)
    dst_r = jnp.concatenate(
        [edge_index[1], jnp.full((e_pad - e,), n, jnp.int32)]
    ).reshape(NS, nch, 1, K)
    ei_r = jnp.concatenate([src_r, dst_r], axis=2)

    u, s = _make_sc_kernel(n, n_pad, nch, d)(x1p, asrc, adst, ei_r)

    u0 = u[0, :n]
    u1 = u[1, :n]
    s0 = s[0].reshape(n_pad)[:n].reshape(n, 1)

    h2 = pl.pallas_call(
        _k2_body,
        grid=(grid,),
        in_specs=[
            pl.BlockSpec((bm, d // 2), lambda i: (i, 0)),
            pl.BlockSpec((bm, d // 2), lambda i: (i, 0)),
            pl.BlockSpec((bm, 1), lambda i: (i, 0)),
            pl.BlockSpec((d, d), lambda i: (0, 0)),
        ],
        out_specs=pl.BlockSpec((bm, d), lambda i: (i, 0)),
        out_shape=jax.ShapeDtypeStruct((n, d), jnp.float32),
    )(u0, u1, s0, W2)
    return h2


# speedup vs baseline: 1.2600x; 1.1277x over previous
import functools
import math

import jax
import jax.numpy as jnp
from jax import lax
from jax.experimental import pallas as pl
from jax.experimental.pallas import tpu as pltpu
from jax.experimental.pallas import tpu_sc as plsc

NC = 2
NS = 16
K = 128
L = 16


def _k1_body(x_ref, w1_ref, att_ref, xp_ref, a2_ref):
    x1 = jnp.dot(x_ref[...], w1_ref[...], preferred_element_type=jnp.float32)
    a2_ref[...] = jnp.dot(x1, att_ref[...], preferred_element_type=jnp.float32)
    vi = jax.lax.bitcast_convert_type(x1, jnp.int32)
    r = vi + 0x7FFF + ((vi >> 16) & 1)
    words = []
    for q in range(x1.shape[1] // 32):
        lo = jax.lax.shift_right_logical(r[:, q * 32:q * 32 + 16], 16)
        hi = r[:, q * 32 + 16:q * 32 + 32] & jnp.int32(-65536)
        words.append(lo | hi)
    xp_ref[...] = jnp.concatenate(words, axis=1)


def _k2_body(u0_ref, u1_ref, s_ref, w2_ref, o_ref):
    s = s_ref[...] + 1e-16
    out = jnp.concatenate([u0_ref[...], u1_ref[...]], axis=1) / s
    h1 = jnp.where(out > 0, out, jnp.exp(out) - 1.0)
    o_ref[...] = jnp.dot(h1, w2_ref[...], preferred_element_type=jnp.float32)


def _make_sc_kernel(n, n_pad, nch, d):
    dh = d // 2
    dw = dh // 2
    assert n % NS == 0
    mesh = plsc.VectorSubcoreMesh(
        core_axis_name="c", subcore_axis_name="s", num_cores=NC, num_subcores=NS
    )
    rows_per_tile = n_pad // NS
    n_zero_copies = rows_per_tile // K
    srow = n_pad // d
    assert srow <= K and srow % L == 0 and rows_per_tile % K == 0

    @functools.partial(
        pl.kernel,
        mesh=mesh,
        compiler_params=pltpu.CompilerParams(
            needs_layout_passes=False, use_tc_tiling_on_sc=False),
        out_type=(
            jax.ShapeDtypeStruct((NC, n_pad, dh), jnp.float32),
            jax.ShapeDtypeStruct((NC, srow, d), jnp.float32),
        ),
        scratch_types=[
            pltpu.VMEM((4, 2, K), jnp.int32),
            pltpu.VMEM((srow, d), jnp.float32),
            pltpu.VMEM((srow, d), jnp.float32),
            pltpu.VMEM((srow, d), jnp.float32),
            pltpu.VMEM((1, K), jnp.float32),
            pltpu.VMEM((2, K, dw), jnp.int32),
            pltpu.VMEM((2, K, dh), jnp.float32),
            pltpu.VMEM((1, srow), jnp.int32),
            pltpu.VMEM_SHARED((n_pad, dw), jnp.int32),
            pltpu.VMEM_SHARED((n_pad, dh), jnp.float32),
            pltpu.VMEM_SHARED((srow, d), jnp.float32),
            pltpu.SemaphoreType.DMA,
            pltpu.SemaphoreType.DMA,
            pltpu.SemaphoreType.DMA,
            pltpu.SemaphoreType.DMA,
            pltpu.SemaphoreType.DMA,
            pltpu.SemaphoreType.DMA,
        ],
    )
    def sc_edge_kernel(x1p_hbm, asrc_hbm, adst_hbm, eir_hbm,
                       u_out, s_out,
                       idx_ring, asrc_v, adst_v, s_loc, pbc, rows_bf, rows_f,
                       rowidx, xt_sh, u_sh, s_sh,
                       gsem0, gsem1, ssem0, ssem1, isem0, isem1):
        gsems = (gsem0, gsem1)
        ssems = (ssem0, ssem1)
        isems = (isem0, isem1)
        cid = lax.axis_index("c")
        sid = lax.axis_index("s")
        zero16 = jnp.zeros((L,), jnp.float32)
        lane16 = lax.iota(jnp.int32, L)
        himask = jnp.full((L,), -65536, jnp.int32)

        def _zrows(i, _):
            for col in range(dh // L):
                rows_f[0, i, pl.ds(col * L, L)] = zero16
            return 0
        lax.fori_loop(0, K, _zrows, 0)

        def _zs(i, _):
            for col in range(d // L):
                s_loc[i, pl.ds(col * L, L)] = zero16
            return 0
        lax.fori_loop(0, srow, _zs, 0)

        for g in range(srow // L):
            rowidx[0, pl.ds(g * L, L)] = lane16 + g * L

        for c in range(n_zero_copies):
            pltpu.sync_copy(rows_f.at[0],
                            u_sh.at[pl.ds(sid * rows_per_tile + c * K, K)])

        @pl.when(sid == 0)
        def _():
            pltpu.sync_copy(s_loc, s_sh)

        pltpu.sync_copy(asrc_hbm, asrc_v)
        pltpu.sync_copy(adst_hbm, adst_v)
        rst = n // NS
        pltpu.sync_copy(
            x1p_hbm.at[pl.ds(sid * rst, rst), pl.ds(cid * dw, dw)],
            xt_sh.at[pl.ds(sid * rst, rst)])

        plsc.subcore_barrier()

        def _wait_idx(p):
            pltpu.make_async_copy(
                eir_hbm.at[sid, 0], idx_ring.at[0], isems[p]).wait()

        def _wait_gather(p):
            pltpu.make_async_copy(
                xt_sh.at[idx_ring.at[0, 0]], rows_bf.at[p], gsems[p]).wait()

        def _wait_scatter(p):
            pltpu.make_async_copy(
                rows_f.at[p], u_sh.at[idx_ring.at[0, 1]], ssems[p]).wait()

        pltpu.async_copy(eir_hbm.at[sid, 0], idx_ring.at[0], isems[0])
        pltpu.async_copy(eir_hbm.at[sid, 1], idx_ring.at[1], isems[1])
        _wait_idx(0)
        pltpu.async_copy(xt_sh.at[idx_ring.at[0, 0]], rows_bf.at[0], gsems[0])

        def ring_body(jo, _):
            for b in range(4):
                j = jo * 4 + b
                bn = (b + 1) % 4
                b2 = (b + 2) % 4
                p = b % 2
                pn = (b + 1) % 2

                @pl.when(j >= 2)
                def _():
                    _wait_scatter(p)

                @pl.when(j + 2 < nch)
                def _():
                    pltpu.async_copy(eir_hbm.at[sid, j + 2], idx_ring.at[b2],
                                     isems[p])

                @pl.when(j + 1 < nch)
                def _():
                    _wait_idx(pn)
                    pltpu.async_copy(
                        xt_sh.at[idx_ring.at[bn, 0]], rows_bf.at[pn],
                        gsems[pn])

                _wait_gather(p)

                for g in range(K // L):
                    si = idx_ring[b, 0, pl.ds(g * L, L)]
                    di = idx_ring[b, 1, pl.ds(g * L, L)]
                    av = plsc.load_gather(asrc_v, [si >> 7, si & (d - 1)])
                    bv = plsc.load_gather(adst_v, [di >> 7, di & (d - 1)])
                    sig = 1.0 / (1.0 + jnp.exp(-(av + bv)))
                    pv = jnp.exp(sig)
                    plsc.addupdate_scatter(s_loc, [di >> 7, di & (d - 1)], pv)
                    pbc[0, pl.ds(g * L, L)] = pv

                def scale_body(g8, _):
                    v = pbc[0, pl.ds(g8 * L, L)]
                    base = g8 * L
                    for r in range(L):
                        pk = v[jnp.full((L,), r, jnp.int32)]
                        k = base + r
                        for q in range(dw // L):
                            w = rows_bf[p, k, pl.ds(q * L, L)]
                            lo = plsc.bitcast(w << 16, jnp.float32)
                            hi = plsc.bitcast(w & himask, jnp.float32)
                            rows_f[p, k, pl.ds(q * 2 * L, L)] = lo * pk
                            rows_f[p, k, pl.ds(q * 2 * L + L, L)] = hi * pk
                    return 0
                lax.fori_loop(0, K // L, scale_body, 0)

                pltpu.async_copy(rows_f.at[p], u_sh.at[idx_ring.at[b, 1]],
                                 ssems[p], add=True)
            return 0

        lax.fori_loop(0, nch // 4, ring_body, 0)

        _wait_scatter(0)
        _wait_scatter(1)

        pltpu.sync_copy(s_loc, s_sh.at[rowidx.at[0]], add=True)
        plsc.subcore_barrier()

        for c in range(n_zero_copies):
            base = sid * rows_per_tile + c * K
            pltpu.sync_copy(u_sh.at[pl.ds(base, K)], u_out.at[cid, pl.ds(base, K)])

        @pl.when(sid == 0)
        def _():
            pltpu.sync_copy(s_sh, s_out.at[cid])

    return sc_edge_kernel


def kernel(x, edge_index, W1, att_src1, att_dst1, W2):
    n, d = x.shape
    e = edge_index.shape[1]
    n_pad = 10240
    bm = 1000
    grid = n // bm

    att_cat = jnp.zeros((d, d), jnp.float32)
    att_cat = att_cat.at[:, 0].set(att_src1.reshape(-1))
    att_cat = att_cat.at[:, 1].set(att_dst1.reshape(-1))

    x1p, a2 = pl.pallas_call(
        _k1_body,
        grid=(grid,),
        in_specs=[
            pl.BlockSpec((bm, d), lambda i: (i, 0)),
            pl.BlockSpec((d, d), lambda i: (0, 0)),
            pl.BlockSpec((d, d), lambda i: (0, 0)),
        ],
        out_specs=[
            pl.BlockSpec((bm, d // 2), lambda i: (i, 0)),
            pl.BlockSpec((bm, d), lambda i: (i, 0)),
        ],
        out_shape=[
            jax.ShapeDtypeStruct((n, d // 2), jnp.int32),
            jax.ShapeDtypeStruct((n, d), jnp.float32),
        ],
    )(x, W1, att_cat)

    asrc = jnp.zeros((n_pad,), jnp.float32).at[:n].set(a2[:, 0]).reshape(n_pad // d, d)
    adst = jnp.zeros((n_pad,), jnp.float32).at[:n].set(a2[:, 1]).reshape(n_pad // d, d)

    nch = math.ceil(e / (NS * K))
    nch = ((nch + 3) // 4) * 4
    e_pad = NS * nch * K
    src_r = jnp.concatenate(
        [edge_index[0], jnp.zeros((e_pad - e,), jnp.int32)]
    ).reshape(NS, nch, 1, K)
    dst_r = jnp.concatenate(
        [edge_index[1], jnp.full((e_pad - e,), n, jnp.int32)]
    ).reshape(NS, nch, 1, K)
    ei_r = jnp.concatenate([src_r, dst_r], axis=2)

    u, s = _make_sc_kernel(n, n_pad, nch, d)(x1p, asrc, adst, ei_r)

    u0 = u[0, :n]
    u1 = u[1, :n]
    s0 = s[0].reshape(n_pad)[:n].reshape(n, 1)

    h2 = pl.pallas_call(
        _k2_body,
        grid=(grid,),
        in_specs=[
            pl.BlockSpec((bm, d // 2), lambda i: (i, 0)),
            pl.BlockSpec((bm, d // 2), lambda i: (i, 0)),
            pl.BlockSpec((bm, 1), lambda i: (i, 0)),
            pl.BlockSpec((d, d), lambda i: (0, 0)),
        ],
        out_specs=pl.BlockSpec((bm, d), lambda i: (i, 0)),
        out_shape=jax.ShapeDtypeStruct((n, d), jnp.float32),
    )(u0, u1, s0, W2)
    return h2


# speedup vs baseline: 1.2927x; 1.0260x over previous
import functools
import math

import jax
import jax.numpy as jnp
from jax import lax
from jax.experimental import pallas as pl
from jax.experimental.pallas import tpu as pltpu
from jax.experimental.pallas import tpu_sc as plsc

NC = 2
NS = 16
K = 128
L = 16


def _k1_body(x_ref, w1_ref, as_ref, ad_ref, xp_ref, a2_ref):
    x1 = jnp.dot(x_ref[...], w1_ref[...], preferred_element_type=jnp.float32)
    asv = jnp.sum(x1 * as_ref[...], axis=1, keepdims=True)
    adv = jnp.sum(x1 * ad_ref[...], axis=1, keepdims=True)
    a2_ref[...] = jnp.concatenate([asv, adv], axis=1)
    vi = jax.lax.bitcast_convert_type(x1, jnp.int32)
    r = vi + 0x7FFF + ((vi >> 16) & 1)
    words = []
    for q in range(x1.shape[1] // 32):
        lo = jax.lax.shift_right_logical(r[:, q * 32:q * 32 + 16], 16)
        hi = r[:, q * 32 + 16:q * 32 + 32] & jnp.int32(-65536)
        words.append(lo | hi)
    xp_ref[...] = jnp.concatenate(words, axis=1)


def _k2_body(u0_ref, u1_ref, s_ref, w2_ref, o_ref):
    s = s_ref[...] + 1e-16
    out = jnp.concatenate([u0_ref[0], u1_ref[0]], axis=1) / s
    h1 = jnp.where(out > 0, out, jnp.exp(out) - 1.0)
    o_ref[...] = jnp.dot(h1, w2_ref[...], preferred_element_type=jnp.float32)


def _make_sc_kernel(n, n_pad, nch, d):
    dh = d // 2
    dw = dh // 2
    assert n % NS == 0
    mesh = plsc.VectorSubcoreMesh(
        core_axis_name="c", subcore_axis_name="s", num_cores=NC, num_subcores=NS
    )
    rows_per_tile = n_pad // NS
    n_zero_copies = rows_per_tile // K
    srow = n_pad // d
    assert srow <= K and srow % L == 0 and rows_per_tile % K == 0

    @functools.partial(
        pl.kernel,
        mesh=mesh,
        compiler_params=pltpu.CompilerParams(
            needs_layout_passes=False, use_tc_tiling_on_sc=False),
        out_type=(
            jax.ShapeDtypeStruct((NC, n_pad, dh), jnp.float32),
            jax.ShapeDtypeStruct((NC, srow, d), jnp.float32),
        ),
        scratch_types=[
            pltpu.VMEM((4, 2, K), jnp.int32),
            pltpu.VMEM((srow, d), jnp.float32),
            pltpu.VMEM((srow, d), jnp.float32),
            pltpu.VMEM((srow, d), jnp.float32),
            pltpu.VMEM((1, K), jnp.float32),
            pltpu.VMEM((2, K, dw), jnp.int32),
            pltpu.VMEM((2, K, dh), jnp.float32),
            pltpu.VMEM((1, srow), jnp.int32),
            pltpu.VMEM_SHARED((n_pad, dw), jnp.int32),
            pltpu.VMEM_SHARED((n_pad, dh), jnp.float32),
            pltpu.VMEM_SHARED((srow, d), jnp.float32),
            pltpu.SemaphoreType.DMA,
            pltpu.SemaphoreType.DMA,
            pltpu.SemaphoreType.DMA,
            pltpu.SemaphoreType.DMA,
            pltpu.SemaphoreType.DMA,
            pltpu.SemaphoreType.DMA,
        ],
    )
    def sc_edge_kernel(x1p_hbm, asrc_hbm, adst_hbm, eir_hbm,
                       u_out, s_out,
                       idx_ring, asrc_v, adst_v, s_loc, pbc, rows_bf, rows_f,
                       rowidx, xt_sh, u_sh, s_sh,
                       gsem0, gsem1, ssem0, ssem1, isem0, isem1):
        gsems = (gsem0, gsem1)
        ssems = (ssem0, ssem1)
        isems = (isem0, isem1)
        cid = lax.axis_index("c")
        sid = lax.axis_index("s")
        zero16 = jnp.zeros((L,), jnp.float32)
        lane16 = lax.iota(jnp.int32, L)
        himask = jnp.full((L,), -65536, jnp.int32)

        def _zrows(i, _):
            for col in range(dh // L):
                rows_f[0, i, pl.ds(col * L, L)] = zero16
            return 0
        lax.fori_loop(0, K, _zrows, 0)

        def _zs(i, _):
            for col in range(d // L):
                s_loc[i, pl.ds(col * L, L)] = zero16
            return 0
        lax.fori_loop(0, srow, _zs, 0)

        for g in range(srow // L):
            rowidx[0, pl.ds(g * L, L)] = lane16 + g * L

        for c in range(n_zero_copies):
            pltpu.sync_copy(rows_f.at[0],
                            u_sh.at[pl.ds(sid * rows_per_tile + c * K, K)])

        @pl.when(sid == 0)
        def _():
            pltpu.sync_copy(s_loc, s_sh)

        pltpu.sync_copy(asrc_hbm, asrc_v)
        pltpu.sync_copy(adst_hbm, adst_v)
        rst = n // NS
        pltpu.sync_copy(
            x1p_hbm.at[pl.ds(sid * rst, rst), pl.ds(cid * dw, dw)],
            xt_sh.at[pl.ds(sid * rst, rst)])

        plsc.subcore_barrier()

        def _wait_idx(p):
            pltpu.make_async_copy(
                eir_hbm.at[sid, 0], idx_ring.at[0], isems[p]).wait()

        def _wait_gather(p):
            pltpu.make_async_copy(
                xt_sh.at[idx_ring.at[0, 0]], rows_bf.at[p], gsems[p]).wait()

        def _wait_scatter(p):
            pltpu.make_async_copy(
                rows_f.at[p], u_sh.at[idx_ring.at[0, 1]], ssems[p]).wait()

        pltpu.async_copy(eir_hbm.at[sid, 0], idx_ring.at[0], isems[0])
        pltpu.async_copy(eir_hbm.at[sid, 1], idx_ring.at[1], isems[1])
        _wait_idx(0)
        pltpu.async_copy(xt_sh.at[idx_ring.at[0, 0]], rows_bf.at[0], gsems[0])

        def ring_body(jo, _):
            for b in range(4):
                j = jo * 4 + b
                bn = (b + 1) % 4
                b2 = (b + 2) % 4
                p = b % 2
                pn = (b + 1) % 2

                @pl.when(j >= 2)
                def _():
                    _wait_scatter(p)

                @pl.when(j + 2 < nch)
                def _():
                    pltpu.async_copy(eir_hbm.at[sid, j + 2], idx_ring.at[b2],
                                     isems[p])

                @pl.when(j + 1 < nch)
                def _():
                    _wait_idx(pn)
                    pltpu.async_copy(
                        xt_sh.at[idx_ring.at[bn, 0]], rows_bf.at[pn],
                        gsems[pn])

                _wait_gather(p)

                for g in range(K // L):
                    si = idx_ring[b, 0, pl.ds(g * L, L)]
                    di = idx_ring[b, 1, pl.ds(g * L, L)]
                    av = plsc.load_gather(asrc_v, [si >> 7, si & (d - 1)])
                    bv = plsc.load_gather(adst_v, [di >> 7, di & (d - 1)])
                    sig = 1.0 / (1.0 + jnp.exp(-(av + bv)))
                    pv = jnp.exp(sig)
                    plsc.addupdate_scatter(s_loc, [di >> 7, di & (d - 1)], pv)
                    pbc[0, pl.ds(g * L, L)] = pv

                def scale_body(g8, _):
                    v = pbc[0, pl.ds(g8 * L, L)]
                    base = g8 * L
                    for r in range(L):
                        pk = v[jnp.full((L,), r, jnp.int32)]
                        k = base + r
                        for q in range(dw // L):
                            w = rows_bf[p, k, pl.ds(q * L, L)]
                            lo = plsc.bitcast(w << 16, jnp.float32)
                            hi = plsc.bitcast(w & himask, jnp.float32)
                            rows_f[p, k, pl.ds(q * 2 * L, L)] = lo * pk
                            rows_f[p, k, pl.ds(q * 2 * L + L, L)] = hi * pk
                    return 0
                lax.fori_loop(0, K // L, scale_body, 0)

                pltpu.async_copy(rows_f.at[p], u_sh.at[idx_ring.at[b, 1]],
                                 ssems[p], add=True)
            return 0

        lax.fori_loop(0, nch // 4, ring_body, 0)

        _wait_scatter(0)
        _wait_scatter(1)

        pltpu.sync_copy(s_loc, s_sh.at[rowidx.at[0]], add=True)
        plsc.subcore_barrier()

        for c in range(n_zero_copies):
            base = sid * rows_per_tile + c * K
            pltpu.sync_copy(u_sh.at[pl.ds(base, K)], u_out.at[cid, pl.ds(base, K)])

        @pl.when(sid == 0)
        def _():
            pltpu.sync_copy(s_sh, s_out.at[cid])

    return sc_edge_kernel


def kernel(x, edge_index, W1, att_src1, att_dst1, W2):
    n, d = x.shape
    e = edge_index.shape[1]
    n_pad = 10240
    bm = 1000
    grid = n // bm

    x1p, a2 = pl.pallas_call(
        _k1_body,
        grid=(grid,),
        in_specs=[
            pl.BlockSpec((bm, d), lambda i: (i, 0)),
            pl.BlockSpec((d, d), lambda i: (0, 0)),
            pl.BlockSpec((1, d), lambda i: (0, 0)),
            pl.BlockSpec((1, d), lambda i: (0, 0)),
        ],
        out_specs=[
            pl.BlockSpec((bm, d // 2), lambda i: (i, 0)),
            pl.BlockSpec((bm, 2), lambda i: (i, 0)),
        ],
        out_shape=[
            jax.ShapeDtypeStruct((n, d // 2), jnp.int32),
            jax.ShapeDtypeStruct((n, 2), jnp.float32),
        ],
    )(x, W1, att_src1.reshape(1, d), att_dst1.reshape(1, d))

    asrc = jnp.zeros((n_pad,), jnp.float32).at[:n].set(a2[:, 0]).reshape(n_pad // d, d)
    adst = jnp.zeros((n_pad,), jnp.float32).at[:n].set(a2[:, 1]).reshape(n_pad // d, d)

    nch = math.ceil(e / (NS * K))
    nch = ((nch + 3) // 4) * 4
    e_pad = NS * nch * K
    src_r = jnp.concatenate(
        [edge_index[0], jnp.zeros((e_pad - e,), jnp.int32)]
    ).reshape(NS, nch, 1, K)
    dst_r = jnp.concatenate(
        [edge_index[1], jnp.full((e_pad - e,), n, jnp.int32)]
    ).reshape(NS, nch, 1, K)
    ei_r = jnp.concatenate([src_r, dst_r], axis=2)

    u, s = _make_sc_kernel(n, n_pad, nch, d)(x1p, asrc, adst, ei_r)

    s0 = s[0].reshape(n_pad)[:n].reshape(n, 1)

    h2 = pl.pallas_call(
        _k2_body,
        grid=(grid,),
        in_specs=[
            pl.BlockSpec((1, bm, d // 2), lambda i: (0, i, 0)),
            pl.BlockSpec((1, bm, d // 2), lambda i: (1, i, 0)),
            pl.BlockSpec((bm, 1), lambda i: (i, 0)),
            pl.BlockSpec((d, d), lambda i: (0, 0)),
        ],
        out_specs=pl.BlockSpec((bm, d), lambda i: (i, 0)),
        out_shape=jax.ShapeDtypeStruct((n, d), jnp.float32),
    )(u, u, s0, W2)
    return h2
